# Initial kernel scaffold; baseline (speedup 1.0000x reference)
import functools

import jax
import jax.numpy as jnp
import numpy as np
from jax import lax
from jax.experimental import pallas as pl
from jax.experimental.pallas import tpu as pltpu
from jax.experimental.pallas import tpu_sc as plsc

B = 16384
F = 26
FIELD_SIZE = 100000
NC, NS = 2, 16
NW = NC * NS
R = B // NW
E = R * F
CHUNK = 128
NCHUNK = E // CHUNK
FIRE = 8

_OFF_FLAT = np.tile(np.arange(F, dtype=np.int32) * FIELD_SIZE, R)


def _body(data_hbm, table_hbm, off_hbm, bias_hbm, out_hbm,
          idx_v, off_v, vals_v, out_v, bias_v, sem):
    wid = lax.axis_index("s") * NC + lax.axis_index("c")
    base_e = wid * E

    pltpu.sync_copy(data_hbm.at[pl.ds(base_e, E)], idx_v)
    pltpu.sync_copy(off_hbm, off_v)
    pltpu.sync_copy(bias_hbm, bias_v)

    def add_body(i, carry):
        sl = pl.ds(i * 16, 16)
        idx_v[sl] = idx_v[sl] + off_v[sl]
        return carry

    lax.fori_loop(0, E // 16, add_body, 0, unroll=8)

    def gather_step(g, carry):
        handles = []
        for j in range(FIRE):
            sl = pl.ds((g * FIRE + j) * CHUNK, CHUNK)
            handles.append(
                pltpu.async_copy(table_hbm.at[idx_v.at[sl]], vals_v.at[sl], sem))
        for h in handles:
            h.wait()
        return carry

    lax.fori_loop(0, NCHUNK // FIRE, gather_step, 0)

    iota26 = lax.iota(jnp.int32, 16) * F
    bias_vec = bias_v[...]

    def red_body(c, carry):
        p0 = iota26 + c * (16 * F)
        acc = bias_vec
        for f in range(F):
            acc = acc + plsc.load_gather(vals_v, [p0 + f])
        out_v[pl.ds(c * 16, 16)] = 1.0 / (1.0 + jnp.exp(-acc))
        return carry

    lax.fori_loop(0, R // 16, red_body, 0)

    pltpu.sync_copy(out_v, out_hbm.at[pl.ds(wid * R, R)])


@jax.jit
def kernel(data, table, bias):
    mesh = plsc.VectorSubcoreMesh(
        core_axis_name="c", subcore_axis_name="s",
        num_cores=NC, num_subcores=NS)
    run = pl.kernel(
        _body,
        out_type=jax.ShapeDtypeStruct((B,), jnp.float32),
        mesh=mesh,
        scratch_types=[
            pltpu.VMEM((E,), jnp.int32),
            pltpu.VMEM((E,), jnp.int32),
            pltpu.VMEM((E,), jnp.float32),
            pltpu.VMEM((R,), jnp.float32),
            pltpu.VMEM((16,), jnp.float32),
            pltpu.SemaphoreType.DMA,
        ],
    )
    off = jnp.asarray(_OFF_FLAT)
    bias16 = jnp.broadcast_to(bias.astype(jnp.float32), (16,))
    return run(data.reshape(-1), table.reshape(-1), off, bias16)


# speedup vs baseline: 1.0666x; 1.0666x over previous
import functools

import jax
import jax.numpy as jnp
import numpy as np
from jax import lax
from jax.experimental import pallas as pl
from jax.experimental.pallas import tpu as pltpu
from jax.experimental.pallas import tpu_sc as plsc

B = 16384
F = 26
FIELD_SIZE = 100000
NC, NS = 2, 16
NW = NC * NS
R = B // NW
E = R * F
CHUNK = 128
NCHUNK = E // CHUNK
FIRE = 8

_OFF_FLAT = np.tile(np.arange(F, dtype=np.int32) * FIELD_SIZE, R)


def _body(data_hbm, table_hbm, off_hbm, bias_hbm, out_hbm,
          idx_v, off_v, vals_v, out_v, bias_v, sem):
    wid = lax.axis_index("s") * NC + lax.axis_index("c")
    base_e = wid * E

    pltpu.sync_copy(data_hbm.at[pl.ds(base_e, E)], idx_v)
    pltpu.sync_copy(off_hbm, off_v)
    pltpu.sync_copy(bias_hbm, bias_v)

    def add_body(i, carry):
        sl = pl.ds(i * 16, 16)
        idx_v[sl] = idx_v[sl] + off_v[sl]
        return carry

    lax.fori_loop(0, E // 16, add_body, 0, unroll=8)

    def gather_step(g, carry):
        handles = []
        for j in range(FIRE):
            sl = pl.ds((g * FIRE + j) * CHUNK, CHUNK)
            handles.append(
                pltpu.async_copy(table_hbm.at[idx_v.at[sl]], vals_v.at[sl], sem))
        for h in handles:
            h.wait()
        return carry

    lax.fori_loop(0, NCHUNK // FIRE, gather_step, 0)

    iota26 = lax.iota(jnp.int32, 16) * F
    bias_vec = bias_v[...]

    def red_body(c, carry):
        p0 = iota26 + c * (16 * F)
        acc = bias_vec
        for f in range(F):
            acc = acc + plsc.load_gather(vals_v, [p0 + f])
        out_v[pl.ds(c * 16, 16)] = 1.0 / (1.0 + jnp.exp(-acc))
        return carry

    lax.fori_loop(0, R // 16, red_body, 0)

    pltpu.sync_copy(out_v, out_hbm.at[pl.ds(wid * R, R)])


@jax.jit
def kernel(data, table, bias):
    mesh = plsc.VectorSubcoreMesh(
        core_axis_name="c", subcore_axis_name="s",
        num_cores=NC, num_subcores=NS)
    run = pl.kernel(
        _body,
        out_type=jax.ShapeDtypeStruct((B,), jnp.float32),
        mesh=mesh,
        compiler_params=pltpu.CompilerParams(needs_layout_passes=False),
        scratch_types=[
            pltpu.VMEM((E,), jnp.int32),
            pltpu.VMEM((E,), jnp.int32),
            pltpu.VMEM((E,), jnp.float32),
            pltpu.VMEM((R,), jnp.float32),
            pltpu.VMEM((16,), jnp.float32),
            pltpu.SemaphoreType.DMA,
        ],
    )
    off = jnp.asarray(_OFF_FLAT)
    bias16 = jnp.broadcast_to(bias.astype(jnp.float32), (16,))
    return run(data.reshape(-1), table.reshape(-1), off, bias16)


# speedup vs baseline: 2.5607x; 2.4009x over previous
import jax
import jax.numpy as jnp
from jax import lax
from jax.experimental import pallas as pl
from jax.experimental.pallas import tpu as pltpu
from jax.experimental.pallas import tpu_sc as plsc

B = 16384
F = 26
FIELD_SIZE = 100000
TOTAL = F * FIELD_SIZE
LO = (TOTAL // 1024) * 1024
HI = TOTAL - LO
NC, NS = 2, 16
NW = NC * NS
R = B // NW
E = R * F
CHUNK = 128
NCHUNK = E // CHUNK
FIRE = 8


def _body(data_hbm, lo_hbm, hi_hbm, bias_hbm, out_hbm,
          idx_v, idxc_v, vals_v, out_v, hi_v, bias_v, sem):
    wid = lax.axis_index("s") * NC + lax.axis_index("c")
    base_e = wid * E

    pltpu.sync_copy(data_hbm.at[pl.ds(base_e, E)], idx_v)
    pltpu.sync_copy(bias_hbm, bias_v)
    pltpu.sync_copy(hi_hbm, hi_v)

    iota = lax.iota(jnp.int32, 16)

    def add_body(i, carry):
        sl = pl.ds(i * 16, 16)
        fld = lax.rem(i * 16 + iota, F)
        idx = idx_v[sl] + fld * FIELD_SIZE
        idx_v[sl] = idx
        idxc_v[sl] = jnp.minimum(idx, LO - 1)
        return carry

    lax.fori_loop(0, E // 16, add_body, 0, unroll=8)

    def gather_step(g, carry):
        handles = []
        for j in range(FIRE):
            sl = pl.ds((g * FIRE + j) * CHUNK, CHUNK)
            handles.append(
                pltpu.async_copy(lo_hbm.at[idxc_v.at[sl]], vals_v.at[sl], sem))
        for h in handles:
            h.wait()
        return carry

    lax.fori_loop(0, NCHUNK // FIRE, gather_step, 0)

    iota26 = iota * F
    bias_vec = bias_v[...]

    def red_body(c, carry):
        p0 = iota26 + c * (16 * F)
        acc = bias_vec
        for f in range(F - 1):
            acc = acc + plsc.load_gather(vals_v, [p0 + f])
        p = p0 + (F - 1)
        v = plsc.load_gather(vals_v, [p])
        io = plsc.load_gather(idx_v, [p])
        hv = plsc.load_gather(hi_v, [jnp.maximum(io - LO, 0)])
        acc = acc + jnp.where(io >= LO, hv, v)
        out_v[pl.ds(c * 16, 16)] = 1.0 / (1.0 + jnp.exp(-acc))
        return carry

    lax.fori_loop(0, R // 16, red_body, 0)

    pltpu.sync_copy(out_v, out_hbm.at[pl.ds(wid * R, R)])


@jax.jit
def kernel(data, table, bias):
    mesh = plsc.VectorSubcoreMesh(
        core_axis_name="c", subcore_axis_name="s",
        num_cores=NC, num_subcores=NS)
    run = pl.kernel(
        _body,
        out_type=jax.ShapeDtypeStruct((B,), jnp.float32),
        mesh=mesh,
        compiler_params=pltpu.CompilerParams(needs_layout_passes=False),
        scratch_types=[
            pltpu.VMEM((E,), jnp.int32),
            pltpu.VMEM((E,), jnp.int32),
            pltpu.VMEM((E,), jnp.float32),
            pltpu.VMEM((R,), jnp.float32),
            pltpu.VMEM((HI,), jnp.float32),
            pltpu.VMEM((16,), jnp.float32),
            pltpu.SemaphoreType.DMA,
        ],
    )
    bias16 = jnp.broadcast_to(bias.astype(jnp.float32), (16,))
    table_lo = lax.slice(table, (0, 0), (LO, 1)).reshape(LO)
    table_hi = lax.slice(table, (LO, 0), (TOTAL, 1)).reshape(HI)
    return run(data.reshape(-1), table_lo, table_hi, bias16)


# speedup vs baseline: 3.0917x; 1.2074x over previous
import jax
import jax.numpy as jnp
from jax import lax
from jax.experimental import pallas as pl
from jax.experimental.pallas import tpu as pltpu
from jax.experimental.pallas import tpu_sc as plsc

B = 16384
F = 26
FIELD_SIZE = 100000
TOTAL = F * FIELD_SIZE
LO = (TOTAL // 1024) * 1024
HI = TOTAL - LO
NC, NS = 2, 16
NW = NC * NS
R = B // NW
E = R * F
CHUNK = 128
NCHUNK = E // CHUNK
FIRE = 8
RC = R // 16


def _body(data_hbm, lo_hbm, hi_hbm, bias_hbm, out_hbm,
          data_v, idx_v, idxc_v, vals_v, out_v, hi_v, bias_v, sem):
    wid = lax.axis_index("s") * NC + lax.axis_index("c")

    pltpu.sync_copy(data_hbm.at[:, pl.ds(wid * R, R)], data_v)
    pltpu.sync_copy(bias_hbm, bias_v)
    pltpu.sync_copy(hi_hbm, hi_v)

    def add_body(j, carry):
        sl = pl.ds(j * 16, 16)
        for f in range(F):
            idx = data_v[f, sl] + f * FIELD_SIZE
            dst = pl.ds(f * R + j * 16, 16)
            idx_v[dst] = idx
            idxc_v[dst] = jnp.minimum(idx, LO - 1)
        return carry

    lax.fori_loop(0, RC, add_body, 0)

    def gather_step(g, carry):
        handles = []
        for j in range(FIRE):
            sl = pl.ds((g * FIRE + j) * CHUNK, CHUNK)
            handles.append(
                pltpu.async_copy(lo_hbm.at[idxc_v.at[sl]], vals_v.at[sl], sem))
        for h in handles:
            h.wait()
        return carry

    lax.fori_loop(0, NCHUNK // FIRE, gather_step, 0)

    bias_vec = bias_v[...]

    def red_body(c, carry):
        rbase = c * 16
        acc = bias_vec
        for f in range(F - 1):
            acc = acc + vals_v[pl.ds(f * R + rbase, 16)]
        sl = pl.ds((F - 1) * R + rbase, 16)
        v = vals_v[sl]
        io = idx_v[sl]
        hv = plsc.load_gather(hi_v, [jnp.maximum(io - LO, 0)])
        acc = acc + jnp.where(io >= LO, hv, v)
        out_v[pl.ds(rbase, 16)] = 1.0 / (1.0 + jnp.exp(-acc))
        return carry

    lax.fori_loop(0, RC, red_body, 0)

    pltpu.sync_copy(out_v, out_hbm.at[pl.ds(wid * R, R)])


@jax.jit
def kernel(data, table, bias):
    mesh = plsc.VectorSubcoreMesh(
        core_axis_name="c", subcore_axis_name="s",
        num_cores=NC, num_subcores=NS)
    run = pl.kernel(
        _body,
        out_type=jax.ShapeDtypeStruct((B,), jnp.float32),
        mesh=mesh,
        compiler_params=pltpu.CompilerParams(needs_layout_passes=False),
        scratch_types=[
            pltpu.VMEM((F, R), jnp.int32),
            pltpu.VMEM((E,), jnp.int32),
            pltpu.VMEM((E,), jnp.int32),
            pltpu.VMEM((E,), jnp.float32),
            pltpu.VMEM((R,), jnp.float32),
            pltpu.VMEM((HI,), jnp.float32),
            pltpu.VMEM((16,), jnp.float32),
            pltpu.SemaphoreType.DMA,
        ],
    )
    bias16 = jnp.broadcast_to(bias.astype(jnp.float32), (16,))
    table_lo = lax.slice(table, (0, 0), (LO, 1)).reshape(LO)
    table_hi = lax.slice(table, (LO, 0), (TOTAL, 1)).reshape(HI)
    return run(data.T, table_lo, table_hi, bias16)


# speedup vs baseline: 3.6816x; 1.1908x over previous
import jax
import jax.numpy as jnp
from jax import lax
from jax.experimental import pallas as pl
from jax.experimental.pallas import tpu as pltpu
from jax.experimental.pallas import tpu_sc as plsc

B = 16384
F = 26
FIELD_SIZE = 100000
TOTAL = F * FIELD_SIZE
LO = (TOTAL // 1024) * 1024
HI = TOTAL - LO
NC, NS = 2, 16
NW = NC * NS
R = B // NW
E = R * F
CHUNK = 512
NCHUNK = E // CHUNK
FIRE = 13
RC = R // 16


def _body(data_hbm, lo_hbm, hi_hbm, bias_hbm, out_hbm,
          data_v, idx_v, idxc_v, vals_v, out_v, hi_v, bias_v, sem):
    wid = lax.axis_index("s") * NC + lax.axis_index("c")

    pltpu.sync_copy(data_hbm.at[:, pl.ds(wid * R, R)], data_v)
    pltpu.sync_copy(bias_hbm, bias_v)
    pltpu.sync_copy(hi_hbm, hi_v)

    def add_body(j, carry):
        sl = pl.ds(j * 16, 16)
        for f in range(F):
            idx = data_v[f, sl] + f * FIELD_SIZE
            dst = pl.ds(f * R + j * 16, 16)
            idx_v[dst] = idx
            idxc_v[dst] = jnp.minimum(idx, LO - 1)
        return carry

    lax.fori_loop(0, RC, add_body, 0)

    handles = []
    for g in range(2):
        for j in range(FIRE):
            sl = pl.ds((g * FIRE + j) * CHUNK, CHUNK)
            handles.append(
                pltpu.async_copy(lo_hbm.at[idxc_v.at[sl]], vals_v.at[sl],
                                 sem.at[g]))
    for h in handles:
        h.wait()

    bias_vec = bias_v[...]

    def red_body(c, carry):
        rbase = c * 16
        acc = bias_vec
        for f in range(F - 1):
            acc = acc + vals_v[pl.ds(f * R + rbase, 16)]
        sl = pl.ds((F - 1) * R + rbase, 16)
        v = vals_v[sl]
        io = idx_v[sl]
        hv = plsc.load_gather(hi_v, [jnp.maximum(io - LO, 0)])
        acc = acc + jnp.where(io >= LO, hv, v)
        out_v[pl.ds(rbase, 16)] = 1.0 / (1.0 + jnp.exp(-acc))
        return carry

    lax.fori_loop(0, RC, red_body, 0)

    pltpu.sync_copy(out_v, out_hbm.at[pl.ds(wid * R, R)])


@jax.jit
def kernel(data, table, bias):
    mesh = plsc.VectorSubcoreMesh(
        core_axis_name="c", subcore_axis_name="s",
        num_cores=NC, num_subcores=NS)
    run = pl.kernel(
        _body,
        out_type=jax.ShapeDtypeStruct((B,), jnp.float32),
        mesh=mesh,
        compiler_params=pltpu.CompilerParams(needs_layout_passes=False),
        scratch_types=[
            pltpu.VMEM((F, R), jnp.int32),
            pltpu.VMEM((E,), jnp.int32),
            pltpu.VMEM((E,), jnp.int32),
            pltpu.VMEM((E,), jnp.float32),
            pltpu.VMEM((R,), jnp.float32),
            pltpu.VMEM((HI,), jnp.float32),
            pltpu.VMEM((16,), jnp.float32),
            pltpu.SemaphoreType.DMA((2,)),
        ],
    )
    bias16 = jnp.broadcast_to(bias.astype(jnp.float32), (16,))
    table_lo = lax.slice(table, (0, 0), (LO, 1)).reshape(LO)
    table_hi = lax.slice(table, (LO, 0), (TOTAL, 1)).reshape(HI)
    return run(data.T, table_lo, table_hi, bias16)


# speedup vs baseline: 4.0398x; 1.0973x over previous
import jax
import jax.numpy as jnp
from jax import lax
from jax.experimental import pallas as pl
from jax.experimental.pallas import tpu as pltpu
from jax.experimental.pallas import tpu_sc as plsc

B = 16384
F = 26
FIELD_SIZE = 100000
TOTAL = F * FIELD_SIZE
LO = (TOTAL // 1024) * 1024
HI = TOTAL - LO
NC, NS = 2, 16
NW = NC * NS
R = B // NW
E = R * F
CHUNK = 512
NCHUNK = E // CHUNK
FIRE = 13
RC = R // 16


def _body(data_hbm, lo_hbm, hi_hbm, bias_hbm, out_hbm,
          data_v, idx_v, idxc_v, vals_v, out_v, hi_v, bias_v, sem):
    wid = lax.axis_index("s") * NC + lax.axis_index("c")

    pltpu.sync_copy(data_hbm.at[:, pl.ds(wid * R, R)], data_v)
    pltpu.sync_copy(bias_hbm, bias_v)
    pltpu.sync_copy(hi_hbm, hi_v)

    handles = []
    for f in range(F):
        def add_body(j, carry, f=f):
            sl = pl.ds(j * 16, 16)
            idx = data_v[f, sl] + f * FIELD_SIZE
            dst = pl.ds(f * R + j * 16, 16)
            idx_v[dst] = idx
            idxc_v[dst] = jnp.minimum(idx, LO - 1)
            return carry

        lax.fori_loop(0, RC, add_body, 0)
        sl = pl.ds(f * R, R)
        handles.append(
            pltpu.async_copy(lo_hbm.at[idxc_v.at[sl]], vals_v.at[sl],
                             sem.at[f]))

    bias_vec = bias_v[...]

    def acc0_body(c, carry):
        sl = pl.ds(c * 16, 16)
        out_v[sl] = vals_v[sl] + bias_vec
        return carry

    handles[0].wait()
    lax.fori_loop(0, RC, acc0_body, 0)

    for f in range(1, F - 1):
        def accf_body(c, carry, f=f):
            sl = pl.ds(c * 16, 16)
            out_v[sl] = out_v[sl] + vals_v[pl.ds(f * R + c * 16, 16)]
            return carry

        handles[f].wait()
        lax.fori_loop(0, RC, accf_body, 0)

    def last_body(c, carry):
        rbase = c * 16
        sl = pl.ds((F - 1) * R + rbase, 16)
        v = vals_v[sl]
        io = idx_v[sl]
        hv = plsc.load_gather(hi_v, [jnp.maximum(io - LO, 0)])
        acc = out_v[pl.ds(rbase, 16)] + jnp.where(io >= LO, hv, v)
        out_v[pl.ds(rbase, 16)] = 1.0 / (1.0 + jnp.exp(-acc))
        return carry

    handles[F - 1].wait()
    lax.fori_loop(0, RC, last_body, 0)

    pltpu.sync_copy(out_v, out_hbm.at[pl.ds(wid * R, R)])


@jax.jit
def kernel(data, table, bias):
    mesh = plsc.VectorSubcoreMesh(
        core_axis_name="c", subcore_axis_name="s",
        num_cores=NC, num_subcores=NS)
    run = pl.kernel(
        _body,
        out_type=jax.ShapeDtypeStruct((B,), jnp.float32),
        mesh=mesh,
        compiler_params=pltpu.CompilerParams(needs_layout_passes=False),
        scratch_types=[
            pltpu.VMEM((F, R), jnp.int32),
            pltpu.VMEM((E,), jnp.int32),
            pltpu.VMEM((E,), jnp.int32),
            pltpu.VMEM((E,), jnp.float32),
            pltpu.VMEM((R,), jnp.float32),
            pltpu.VMEM((HI,), jnp.float32),
            pltpu.VMEM((16,), jnp.float32),
            pltpu.SemaphoreType.DMA((F,)),
        ],
    )
    bias16 = jnp.broadcast_to(bias.astype(jnp.float32), (16,))
    table_lo = lax.slice(table, (0, 0), (LO, 1)).reshape(LO)
    table_hi = lax.slice(table, (LO, 0), (TOTAL, 1)).reshape(HI)
    return run(data.T, table_lo, table_hi, bias16)
